# baseline (speedup 1.0000x reference)
import functools

import jax
import jax.numpy as jnp
from jax import lax
from jax.experimental import pallas as pl
from jax.experimental.pallas import tpu as pltpu
from jax.experimental.pallas import tpu_sc as plsc

NC, NS = 2, 16
NW = NC * NS
B = 16384
BW = B // NW
CM = 128
DG = 64
DP = 128
DM = 256
V = 1000000
CB = 4096
NPB = (V + CB - 1) // CB
PB = NPB * (CB // 2)


def _tc_pack_body(ugT, igT, wdiag, ident, pu, pi):
    f32 = jnp.float32
    hp = jax.lax.Precision.HIGHEST
    dn = (((0,), (0,)), ((), ()))
    tu = jax.lax.dot_general(ugT[...], wdiag[...], dn, precision=hp,
                             preferred_element_type=f32)
    ti = jax.lax.dot_general(igT[...], ident[...], dn, precision=hp,
                             preferred_element_type=f32)
    pu[...] = jnp.concatenate([tu[:CB // 2], tu[CB // 2:]], axis=1)
    pi[...] = jnp.concatenate([ti[:CB // 2], ti[CB // 2:]], axis=1)


def _tc_pack(ugT, igT, wdiag, ident):
    f32 = jnp.float32
    return pl.pallas_call(
        _tc_pack_body,
        grid=(NPB,),
        in_specs=[
            pl.BlockSpec((DG, CB), lambda i: (0, i)),
            pl.BlockSpec((DG, CB), lambda i: (0, i)),
            pl.BlockSpec((DG, DG), lambda i: (0, 0)),
            pl.BlockSpec((DG, DG), lambda i: (0, 0)),
        ],
        out_specs=[
            pl.BlockSpec((CB // 2, DP), lambda i: (i, 0)),
            pl.BlockSpec((CB // 2, DP), lambda i: (i, 0)),
        ],
        out_shape=[
            jax.ShapeDtypeStruct((PB, DP), f32),
            jax.ShapeDtypeStruct((PB, DP), f32),
        ],
    )(ugT, igT, wdiag, ident)


def _sc_mlp_body(user_hbm, item_hbm, um_tab, im_tab, um_out, im_out,
                 idx_u, idx_i, um_v, im_v, s1, s2):
    wid = lax.axis_index("s") * NC + lax.axis_index("c")
    for c in range(BW // CM):
        base = wid * BW + c * CM
        pltpu.sync_copy(user_hbm.at[pl.ds(base, CM)], idx_u)
        pltpu.sync_copy(item_hbm.at[pl.ds(base, CM)], idx_i)
        cp_um = pltpu.async_copy(um_tab.at[idx_u], um_v, s1)
        cp_im = pltpu.async_copy(im_tab.at[idx_i], im_v, s2)
        cp_um.wait()
        pltpu.sync_copy(um_v, um_out.at[pl.ds(base, CM)])
        cp_im.wait()
        pltpu.sync_copy(im_v, im_out.at[pl.ds(base, CM)])


def _sc_gmf_body(uk_hbm, ik_hbm, pu_tab, pi_tab, pu_out, pi_out,
                 idx_u, idx_i, pu_v, pi_v, s1, s2):
    wid = lax.axis_index("s") * NC + lax.axis_index("c")
    for c in range(BW // CM):
        base = wid * BW + c * CM
        pltpu.sync_copy(uk_hbm.at[pl.ds(base, CM)], idx_u)
        pltpu.sync_copy(ik_hbm.at[pl.ds(base, CM)], idx_i)
        cp_pu = pltpu.async_copy(pu_tab.at[idx_u], pu_v, s1)
        cp_pi = pltpu.async_copy(pi_tab.at[idx_i], pi_v, s2)
        cp_pu.wait()
        pltpu.sync_copy(pu_v, pu_out.at[pl.ds(base, CM)])
        cp_pi.wait()
        pltpu.sync_copy(pi_v, pi_out.at[pl.ds(base, CM)])


def _mesh():
    return plsc.VectorSubcoreMesh(core_axis_name="c", subcore_axis_name="s",
                                  num_cores=NC, num_subcores=NS)


def _sc_gather_mlp(user, item, um_tab, im_tab):
    f32 = jnp.float32
    fn = pl.kernel(
        _sc_mlp_body,
        out_type=[
            jax.ShapeDtypeStruct((B, DM), f32),
            jax.ShapeDtypeStruct((B, DM), f32),
        ],
        mesh=_mesh(),
        scratch_types=[
            pltpu.VMEM((CM,), jnp.int32),
            pltpu.VMEM((CM,), jnp.int32),
            pltpu.VMEM((CM, DM), f32),
            pltpu.VMEM((CM, DM), f32),
            pltpu.SemaphoreType.DMA,
            pltpu.SemaphoreType.DMA,
        ],
    )
    return fn(user, item, um_tab, im_tab)


def _sc_gather_gmf(uk, ik, pu_tab, pi_tab):
    f32 = jnp.float32
    fn = pl.kernel(
        _sc_gmf_body,
        out_type=[
            jax.ShapeDtypeStruct((B, DP), f32),
            jax.ShapeDtypeStruct((B, DP), f32),
        ],
        mesh=_mesh(),
        scratch_types=[
            pltpu.VMEM((CM,), jnp.int32),
            pltpu.VMEM((CM,), jnp.int32),
            pltpu.VMEM((CM, DP), f32),
            pltpu.VMEM((CM, DP), f32),
            pltpu.SemaphoreType.DMA,
            pltpu.SemaphoreType.DMA,
        ],
    )
    return fn(uk, ik, pu_tab, pi_tab)


TB = 1024


def _tc_mlp_body(pur, pir, uh, ih, um, im, w1a, w1b, b1, w2, b2, w3, b3,
                 wpb, bp, out):
    f32 = jnp.float32
    hp = jax.lax.Precision.HIGHEST
    h = jnp.dot(um[...], w1a[...], precision=hp, preferred_element_type=f32)
    h += jnp.dot(im[...], w1b[...], precision=hp, preferred_element_type=f32)
    h = jnp.maximum(h + b1[...], 0.0)
    h = jnp.maximum(
        jnp.dot(h, w2[...], precision=hp, preferred_element_type=f32)
        + b2[...], 0.0)
    h = jnp.maximum(
        jnp.dot(h, w3[...], precision=hp, preferred_element_type=f32)
        + b3[...], 0.0)
    pu = pur[...]
    pi = pir[...]
    ug = pu[:, :DG] + (pu[:, DG:] - pu[:, :DG]) * uh[...]
    ig = pi[:, :DG] + (pi[:, DG:] - pi[:, :DG]) * ih[...]
    pred = (jnp.sum(ug * ig, axis=1) + jnp.sum(h * wpb[...], axis=1)
            + bp[0, 0])
    out[...] = pred


def _tc_mlp(pur, pir, uh, ih, um, im, W1, b1, W2, b2, W3, b3, Wp, bp):
    f32 = jnp.float32
    w1a, w1b = W1[:DM], W1[DM:]
    wpb = Wp[DG:, 0].reshape(1, DG)
    grid = (B // TB,)
    full = lambda i: (0, 0)
    return pl.pallas_call(
        _tc_mlp_body,
        grid=grid,
        in_specs=[
            pl.BlockSpec((TB, DP), lambda i: (i, 0)),
            pl.BlockSpec((TB, DP), lambda i: (i, 0)),
            pl.BlockSpec((TB, 1), lambda i: (i, 0)),
            pl.BlockSpec((TB, 1), lambda i: (i, 0)),
            pl.BlockSpec((TB, DM), lambda i: (i, 0)),
            pl.BlockSpec((TB, DM), lambda i: (i, 0)),
            pl.BlockSpec((DM, 256), full),
            pl.BlockSpec((DM, 256), full),
            pl.BlockSpec((1, 256), full),
            pl.BlockSpec((256, 128), full),
            pl.BlockSpec((1, 128), full),
            pl.BlockSpec((128, DG), full),
            pl.BlockSpec((1, DG), full),
            pl.BlockSpec((1, DG), full),
            pl.BlockSpec((1, 1), full),
        ],
        out_specs=pl.BlockSpec((TB,), lambda i: (i,)),
        out_shape=jax.ShapeDtypeStruct((B,), f32),
    )(pur, pir, uh, ih, um, im, w1a, w1b, b1.reshape(1, 256), W2,
      b2.reshape(1, 128), W3, b3.reshape(1, DG), wpb, bp.reshape(1, 1))


def kernel(user, item, embed_user_GMF, embed_item_GMF, embed_user_MLP,
           embed_item_MLP, W1, b1, W2, b2, W3, b3, Wp, bp):
    f32 = jnp.float32
    um, im = _sc_gather_mlp(user, item, embed_user_MLP, embed_item_MLP)

    ugT = embed_user_GMF.T
    igT = embed_item_GMF.T
    wdiag = jnp.diag(Wp[:DG, 0])
    ident = jnp.eye(DG, dtype=f32)
    pu_tab, pi_tab = _tc_pack(ugT, igT, wdiag, ident)

    blk = jax.lax.shift_right_logical(user, 12)
    uk = blk * (CB // 2) + jax.lax.bitwise_and(user, (CB // 2) - 1)
    uh = jax.lax.convert_element_type(
        jax.lax.bitwise_and(jax.lax.shift_right_logical(user, 11), 1),
        f32).reshape(B, 1)
    blk = jax.lax.shift_right_logical(item, 12)
    ik = blk * (CB // 2) + jax.lax.bitwise_and(item, (CB // 2) - 1)
    ih = jax.lax.convert_element_type(
        jax.lax.bitwise_and(jax.lax.shift_right_logical(item, 11), 1),
        f32).reshape(B, 1)

    pur, pir = _sc_gather_gmf(uk, ik, pu_tab, pi_tab)
    return _tc_mlp(pur, pir, uh, ih, um, im, W1, b1, W2, b2, W3, b3, Wp, bp)


# speedup vs baseline: 1.8429x; 1.8429x over previous
import functools

import jax
import jax.numpy as jnp
from jax import lax
from jax.experimental import pallas as pl
from jax.experimental.pallas import tpu as pltpu
from jax.experimental.pallas import tpu_sc as plsc

NC, NS = 2, 16
NW = NC * NS
B = 16384
BW = B // NW
CM = 128
DG = 64
DP = 128
DM = 256
V = 1000000
CB = 4096
NPB = (V + CB - 1) // CB
PB = NPB * (CB // 2)


def _tc_pack_body(ugT, igT, wrow, pu, pi):
    tu = ugT[...].T * wrow[...]
    ti = igT[...].T
    pu[...] = jnp.concatenate([tu[:CB // 2], tu[CB // 2:]], axis=1)
    pi[...] = jnp.concatenate([ti[:CB // 2], ti[CB // 2:]], axis=1)


def _tc_pack(ugT, igT, wrow):
    f32 = jnp.float32
    return pl.pallas_call(
        _tc_pack_body,
        grid=(NPB,),
        in_specs=[
            pl.BlockSpec((DG, CB), lambda i: (0, i)),
            pl.BlockSpec((DG, CB), lambda i: (0, i)),
            pl.BlockSpec((1, DG), lambda i: (0, 0)),
        ],
        out_specs=[
            pl.BlockSpec((CB // 2, DP), lambda i: (i, 0)),
            pl.BlockSpec((CB // 2, DP), lambda i: (i, 0)),
        ],
        out_shape=[
            jax.ShapeDtypeStruct((PB, DP), f32),
            jax.ShapeDtypeStruct((PB, DP), f32),
        ],
    )(ugT, igT, wrow)


def _sc_mlp_body(user_hbm, item_hbm, um_tab, im_tab, um_out, im_out,
                 idx_u, idx_i, um_v, im_v, s1, s2):
    wid = lax.axis_index("s") * NC + lax.axis_index("c")
    for c in range(BW // CM):
        base = wid * BW + c * CM
        pltpu.sync_copy(user_hbm.at[pl.ds(base, CM)], idx_u)
        pltpu.sync_copy(item_hbm.at[pl.ds(base, CM)], idx_i)
        cp_um = pltpu.async_copy(um_tab.at[idx_u], um_v, s1)
        cp_im = pltpu.async_copy(im_tab.at[idx_i], im_v, s2)
        cp_um.wait()
        pltpu.sync_copy(um_v, um_out.at[pl.ds(base, CM)])
        cp_im.wait()
        pltpu.sync_copy(im_v, im_out.at[pl.ds(base, CM)])


def _sc_gmf_body(uk_hbm, ik_hbm, pu_tab, pi_tab, pu_out, pi_out,
                 idx_u, idx_i, pu_v, pi_v, s1, s2):
    wid = lax.axis_index("s") * NC + lax.axis_index("c")
    for c in range(BW // CM):
        base = wid * BW + c * CM
        pltpu.sync_copy(uk_hbm.at[pl.ds(base, CM)], idx_u)
        pltpu.sync_copy(ik_hbm.at[pl.ds(base, CM)], idx_i)
        cp_pu = pltpu.async_copy(pu_tab.at[idx_u], pu_v, s1)
        cp_pi = pltpu.async_copy(pi_tab.at[idx_i], pi_v, s2)
        cp_pu.wait()
        pltpu.sync_copy(pu_v, pu_out.at[pl.ds(base, CM)])
        cp_pi.wait()
        pltpu.sync_copy(pi_v, pi_out.at[pl.ds(base, CM)])


def _mesh():
    return plsc.VectorSubcoreMesh(core_axis_name="c", subcore_axis_name="s",
                                  num_cores=NC, num_subcores=NS)


def _sc_gather_mlp(user, item, um_tab, im_tab):
    f32 = jnp.float32
    fn = pl.kernel(
        _sc_mlp_body,
        out_type=[
            jax.ShapeDtypeStruct((B, DM), f32),
            jax.ShapeDtypeStruct((B, DM), f32),
        ],
        mesh=_mesh(),
        scratch_types=[
            pltpu.VMEM((CM,), jnp.int32),
            pltpu.VMEM((CM,), jnp.int32),
            pltpu.VMEM((CM, DM), f32),
            pltpu.VMEM((CM, DM), f32),
            pltpu.SemaphoreType.DMA,
            pltpu.SemaphoreType.DMA,
        ],
    )
    return fn(user, item, um_tab, im_tab)


def _sc_gather_gmf(uk, ik, pu_tab, pi_tab):
    f32 = jnp.float32
    fn = pl.kernel(
        _sc_gmf_body,
        out_type=[
            jax.ShapeDtypeStruct((B, DP), f32),
            jax.ShapeDtypeStruct((B, DP), f32),
        ],
        mesh=_mesh(),
        scratch_types=[
            pltpu.VMEM((CM,), jnp.int32),
            pltpu.VMEM((CM,), jnp.int32),
            pltpu.VMEM((CM, DP), f32),
            pltpu.VMEM((CM, DP), f32),
            pltpu.SemaphoreType.DMA,
            pltpu.SemaphoreType.DMA,
        ],
    )
    return fn(uk, ik, pu_tab, pi_tab)


TB = 1024


def _tc_mlp_body(pur, pir, uh, ih, um, im, w1a, w1b, b1, w2, b2, w3, b3,
                 wpb, bp, out):
    f32 = jnp.float32
    hp = jax.lax.Precision.HIGHEST
    h = jnp.dot(um[...], w1a[...], precision=hp, preferred_element_type=f32)
    h += jnp.dot(im[...], w1b[...], precision=hp, preferred_element_type=f32)
    h = jnp.maximum(h + b1[...], 0.0)
    h = jnp.maximum(
        jnp.dot(h, w2[...], precision=hp, preferred_element_type=f32)
        + b2[...], 0.0)
    h = jnp.maximum(
        jnp.dot(h, w3[...], precision=hp, preferred_element_type=f32)
        + b3[...], 0.0)
    pu = pur[...]
    pi = pir[...]
    ug = pu[:, :DG] + (pu[:, DG:] - pu[:, :DG]) * uh[...]
    ig = pi[:, :DG] + (pi[:, DG:] - pi[:, :DG]) * ih[...]
    pred = (jnp.sum(ug * ig, axis=1) + jnp.sum(h * wpb[...], axis=1)
            + bp[0, 0])
    out[...] = pred


def _tc_mlp(pur, pir, uh, ih, um, im, W1, b1, W2, b2, W3, b3, Wp, bp):
    f32 = jnp.float32
    w1a, w1b = W1[:DM], W1[DM:]
    wpb = Wp[DG:, 0].reshape(1, DG)
    grid = (B // TB,)
    full = lambda i: (0, 0)
    return pl.pallas_call(
        _tc_mlp_body,
        grid=grid,
        in_specs=[
            pl.BlockSpec((TB, DP), lambda i: (i, 0)),
            pl.BlockSpec((TB, DP), lambda i: (i, 0)),
            pl.BlockSpec((TB, 1), lambda i: (i, 0)),
            pl.BlockSpec((TB, 1), lambda i: (i, 0)),
            pl.BlockSpec((TB, DM), lambda i: (i, 0)),
            pl.BlockSpec((TB, DM), lambda i: (i, 0)),
            pl.BlockSpec((DM, 256), full),
            pl.BlockSpec((DM, 256), full),
            pl.BlockSpec((1, 256), full),
            pl.BlockSpec((256, 128), full),
            pl.BlockSpec((1, 128), full),
            pl.BlockSpec((128, DG), full),
            pl.BlockSpec((1, DG), full),
            pl.BlockSpec((1, DG), full),
            pl.BlockSpec((1, 1), full),
        ],
        out_specs=pl.BlockSpec((TB,), lambda i: (i,)),
        out_shape=jax.ShapeDtypeStruct((B,), f32),
    )(pur, pir, uh, ih, um, im, w1a, w1b, b1.reshape(1, 256), W2,
      b2.reshape(1, 128), W3, b3.reshape(1, DG), wpb, bp.reshape(1, 1))


def kernel(user, item, embed_user_GMF, embed_item_GMF, embed_user_MLP,
           embed_item_MLP, W1, b1, W2, b2, W3, b3, Wp, bp):
    f32 = jnp.float32
    um, im = _sc_gather_mlp(user, item, embed_user_MLP, embed_item_MLP)

    ugT = embed_user_GMF.T
    igT = embed_item_GMF.T
    pu_tab, pi_tab = _tc_pack(ugT, igT, Wp[:DG, 0].reshape(1, DG))

    blk = jax.lax.shift_right_logical(user, 12)
    uk = blk * (CB // 2) + jax.lax.bitwise_and(user, (CB // 2) - 1)
    uh = jax.lax.convert_element_type(
        jax.lax.bitwise_and(jax.lax.shift_right_logical(user, 11), 1),
        f32).reshape(B, 1)
    blk = jax.lax.shift_right_logical(item, 12)
    ik = blk * (CB // 2) + jax.lax.bitwise_and(item, (CB // 2) - 1)
    ih = jax.lax.convert_element_type(
        jax.lax.bitwise_and(jax.lax.shift_right_logical(item, 11), 1),
        f32).reshape(B, 1)

    pur, pir = _sc_gather_gmf(uk, ik, pu_tab, pi_tab)
    return _tc_mlp(pur, pir, uh, ih, um, im, W1, b1, W2, b2, W3, b3, Wp, bp)


# speedup vs baseline: 2.4501x; 1.3295x over previous
import functools

import jax
import jax.numpy as jnp
from jax import lax
from jax.experimental import pallas as pl
from jax.experimental.pallas import tpu as pltpu
from jax.experimental.pallas import tpu_sc as plsc

NC, NS = 2, 16
NW = NC * NS
B = 16384
BW = B // NW
CM = 128
DG = 64
DP = 128
DM = 256
V = 1000000
CB = 8192
NPB = (V + CB - 1) // CB
PB = NPB * (CB // 2)


def _tc_pack_body(ugT, igT, wrow, pu, pi):
    tu = ugT[...].T * wrow[...]
    ti = igT[...].T
    pu[...] = jnp.concatenate([tu[:CB // 2], tu[CB // 2:]], axis=1)
    pi[...] = jnp.concatenate([ti[:CB // 2], ti[CB // 2:]], axis=1)


def _tc_pack(ugT, igT, wrow):
    f32 = jnp.float32
    return pl.pallas_call(
        _tc_pack_body,
        grid=(NPB,),
        in_specs=[
            pl.BlockSpec((DG, CB), lambda i: (0, i)),
            pl.BlockSpec((DG, CB), lambda i: (0, i)),
            pl.BlockSpec((1, DG), lambda i: (0, 0)),
        ],
        out_specs=[
            pl.BlockSpec((CB // 2, DP), lambda i: (i, 0)),
            pl.BlockSpec((CB // 2, DP), lambda i: (i, 0)),
        ],
        out_shape=[
            jax.ShapeDtypeStruct((PB, DP), f32),
            jax.ShapeDtypeStruct((PB, DP), f32),
        ],
    )(ugT, igT, wrow)


def _sc_mlp_body(user_hbm, item_hbm, um_tab, im_tab, um_out, im_out,
                 idx_u, idx_i, um_v, im_v, s1, s2):
    wid = lax.axis_index("s") * NC + lax.axis_index("c")
    for c in range(BW // CM):
        base = wid * BW + c * CM
        pltpu.sync_copy(user_hbm.at[pl.ds(base, CM)], idx_u)
        pltpu.sync_copy(item_hbm.at[pl.ds(base, CM)], idx_i)
        cp_um = pltpu.async_copy(um_tab.at[idx_u], um_v, s1)
        cp_im = pltpu.async_copy(im_tab.at[idx_i], im_v, s2)
        cp_um.wait()
        pltpu.sync_copy(um_v, um_out.at[pl.ds(base, CM)])
        cp_im.wait()
        pltpu.sync_copy(im_v, im_out.at[pl.ds(base, CM)])


def _sc_gmf_body(uk_hbm, ik_hbm, pu_tab, pi_tab, dep_hbm, pu_out, pi_out,
                 idx_u, idx_i, pu_v, pi_v, s1, s2):
    wid = lax.axis_index("s") * NC + lax.axis_index("c")
    for c in range(BW // CM):
        base = wid * BW + c * CM
        pltpu.sync_copy(uk_hbm.at[pl.ds(base, CM)], idx_u)
        pltpu.sync_copy(ik_hbm.at[pl.ds(base, CM)], idx_i)
        cp_pu = pltpu.async_copy(pu_tab.at[idx_u], pu_v, s1)
        cp_pi = pltpu.async_copy(pi_tab.at[idx_i], pi_v, s2)
        cp_pu.wait()
        pltpu.sync_copy(pu_v, pu_out.at[pl.ds(base, CM)])
        cp_pi.wait()
        pltpu.sync_copy(pi_v, pi_out.at[pl.ds(base, CM)])


def _mesh():
    return plsc.VectorSubcoreMesh(core_axis_name="c", subcore_axis_name="s",
                                  num_cores=NC, num_subcores=NS)


def _sc_gather_mlp(user, item, um_tab, im_tab):
    f32 = jnp.float32
    fn = pl.kernel(
        _sc_mlp_body,
        out_type=[
            jax.ShapeDtypeStruct((B, DM), f32),
            jax.ShapeDtypeStruct((B, DM), f32),
        ],
        mesh=_mesh(),
        scratch_types=[
            pltpu.VMEM((CM,), jnp.int32),
            pltpu.VMEM((CM,), jnp.int32),
            pltpu.VMEM((CM, DM), f32),
            pltpu.VMEM((CM, DM), f32),
            pltpu.SemaphoreType.DMA,
            pltpu.SemaphoreType.DMA,
        ],
    )
    return fn(user, item, um_tab, im_tab)


def _sc_gather_gmf(uk, ik, pu_tab, pi_tab, dep):
    f32 = jnp.float32
    fn = pl.kernel(
        _sc_gmf_body,
        out_type=[
            jax.ShapeDtypeStruct((B, DP), f32),
            jax.ShapeDtypeStruct((B, DP), f32),
        ],
        mesh=_mesh(),
        scratch_types=[
            pltpu.VMEM((CM,), jnp.int32),
            pltpu.VMEM((CM,), jnp.int32),
            pltpu.VMEM((CM, DP), f32),
            pltpu.VMEM((CM, DP), f32),
            pltpu.SemaphoreType.DMA,
            pltpu.SemaphoreType.DMA,
        ],
    )
    return fn(uk, ik, pu_tab, pi_tab, dep)


TB = 1024


def _tc_mlp_body(pur, pir, uh, ih, um, im, w1a, w1b, b1, w2, b2, w3, b3,
                 wpb, bp, out):
    f32 = jnp.float32
    hp = jax.lax.Precision.DEFAULT
    h = jnp.dot(um[...], w1a[...], precision=hp, preferred_element_type=f32)
    h += jnp.dot(im[...], w1b[...], precision=hp, preferred_element_type=f32)
    h = jnp.maximum(h + b1[...], 0.0)
    h = jnp.maximum(
        jnp.dot(h, w2[...], precision=hp, preferred_element_type=f32)
        + b2[...], 0.0)
    h = jnp.maximum(
        jnp.dot(h, w3[...], precision=hp, preferred_element_type=f32)
        + b3[...], 0.0)
    pu = pur[...]
    pi = pir[...]
    ug = pu[:, :DG] + (pu[:, DG:] - pu[:, :DG]) * uh[...]
    ig = pi[:, :DG] + (pi[:, DG:] - pi[:, :DG]) * ih[...]
    pred = (jnp.sum(ug * ig, axis=1) + jnp.sum(h * wpb[...], axis=1)
            + bp[0, 0])
    out[...] = pred


def _tc_mlp(pur, pir, uh, ih, um, im, W1, b1, W2, b2, W3, b3, Wp, bp):
    f32 = jnp.float32
    w1a, w1b = W1[:DM], W1[DM:]
    wpb = Wp[DG:, 0].reshape(1, DG)
    grid = (B // TB,)
    full = lambda i: (0, 0)
    return pl.pallas_call(
        _tc_mlp_body,
        grid=grid,
        in_specs=[
            pl.BlockSpec((TB, DP), lambda i: (i, 0)),
            pl.BlockSpec((TB, DP), lambda i: (i, 0)),
            pl.BlockSpec((TB, 1), lambda i: (i, 0)),
            pl.BlockSpec((TB, 1), lambda i: (i, 0)),
            pl.BlockSpec((TB, DM), lambda i: (i, 0)),
            pl.BlockSpec((TB, DM), lambda i: (i, 0)),
            pl.BlockSpec((DM, 256), full),
            pl.BlockSpec((DM, 256), full),
            pl.BlockSpec((1, 256), full),
            pl.BlockSpec((256, 128), full),
            pl.BlockSpec((1, 128), full),
            pl.BlockSpec((128, DG), full),
            pl.BlockSpec((1, DG), full),
            pl.BlockSpec((1, DG), full),
            pl.BlockSpec((1, 1), full),
        ],
        out_specs=pl.BlockSpec((TB,), lambda i: (i,)),
        out_shape=jax.ShapeDtypeStruct((B,), f32),
    )(pur, pir, uh, ih, um, im, w1a, w1b, b1.reshape(1, 256), W2,
      b2.reshape(1, 128), W3, b3.reshape(1, DG), wpb, bp.reshape(1, 1))


def kernel(user, item, embed_user_GMF, embed_item_GMF, embed_user_MLP,
           embed_item_MLP, W1, b1, W2, b2, W3, b3, Wp, bp):
    f32 = jnp.float32
    um, im = _sc_gather_mlp(user, item, embed_user_MLP, embed_item_MLP)

    ugT = embed_user_GMF.T
    igT = embed_item_GMF.T
    pu_tab, pi_tab = _tc_pack(ugT, igT, Wp[:DG, 0].reshape(1, DG))

    sh = CB.bit_length() - 1
    blk = jax.lax.shift_right_logical(user, sh)
    uk = blk * (CB // 2) + jax.lax.bitwise_and(user, (CB // 2) - 1)
    uh = jax.lax.convert_element_type(
        jax.lax.bitwise_and(jax.lax.shift_right_logical(user, sh - 1), 1),
        f32).reshape(B, 1)
    blk = jax.lax.shift_right_logical(item, sh)
    ik = blk * (CB // 2) + jax.lax.bitwise_and(item, (CB // 2) - 1)
    ih = jax.lax.convert_element_type(
        jax.lax.bitwise_and(jax.lax.shift_right_logical(item, sh - 1), 1),
        f32).reshape(B, 1)

    pur, pir = _sc_gather_gmf(uk, ik, pu_tab, pi_tab, um)
    return _tc_mlp(pur, pir, uh, ih, um, im, W1, b1, W2, b2, W3, b3, Wp, bp)


# speedup vs baseline: 2.9324x; 1.1968x over previous
import functools

import jax
import jax.numpy as jnp
from jax import lax
from jax.experimental import pallas as pl
from jax.experimental.pallas import tpu as pltpu
from jax.experimental.pallas import tpu_sc as plsc

NC, NS = 2, 16
NW = NC * NS
B = 16384
BW = B // NW
CM = 128
DG = 64
DP = 128
DM = 256
V = 1000000
CB = 8192
NPB = (V + CB - 1) // CB
PB = NPB * (CB // 4)


def _pack_pair(lo, hi):
    u = jnp.uint32
    lo16 = jax.lax.bitcast_convert_type(lo.astype(jnp.bfloat16), jnp.uint16)
    hi16 = jax.lax.bitcast_convert_type(hi.astype(jnp.bfloat16), jnp.uint16)
    w = (lo16.astype(u) | jax.lax.shift_left(hi16.astype(u), jnp.uint32(16)))
    return jax.lax.bitcast_convert_type(w, jnp.float32)


def _tc_pack_body(ugT, igT, wrow, pu, pi):
    tu = ugT[...].T * wrow[...]
    ti = igT[...].T
    H = CB // 4
    pru = _pack_pair(tu[:CB // 2], tu[CB // 2:])
    pri = _pack_pair(ti[:CB // 2], ti[CB // 2:])
    pu[...] = jnp.concatenate([pru[:H], pru[H:]], axis=1)
    pi[...] = jnp.concatenate([pri[:H], pri[H:]], axis=1)


def _tc_pack(ugT, igT, wrow):
    f32 = jnp.float32
    return pl.pallas_call(
        _tc_pack_body,
        grid=(NPB,),
        in_specs=[
            pl.BlockSpec((DG, CB), lambda i: (0, i)),
            pl.BlockSpec((DG, CB), lambda i: (0, i)),
            pl.BlockSpec((1, DG), lambda i: (0, 0)),
        ],
        out_specs=[
            pl.BlockSpec((CB // 4, DP), lambda i: (i, 0)),
            pl.BlockSpec((CB // 4, DP), lambda i: (i, 0)),
        ],
        out_shape=[
            jax.ShapeDtypeStruct((PB, DP), f32),
            jax.ShapeDtypeStruct((PB, DP), f32),
        ],
    )(ugT, igT, wrow)


def _sc_mlp_body(user_hbm, item_hbm, um_tab, im_tab, um_out, im_out,
                 idx_u, idx_i, um_v, im_v, s1, s2):
    wid = lax.axis_index("s") * NC + lax.axis_index("c")
    for c in range(BW // CM):
        base = wid * BW + c * CM
        pltpu.sync_copy(user_hbm.at[pl.ds(base, CM)], idx_u)
        pltpu.sync_copy(item_hbm.at[pl.ds(base, CM)], idx_i)
        cp_um = pltpu.async_copy(um_tab.at[idx_u], um_v, s1)
        cp_im = pltpu.async_copy(im_tab.at[idx_i], im_v, s2)
        cp_um.wait()
        pltpu.sync_copy(um_v, um_out.at[pl.ds(base, CM)])
        cp_im.wait()
        pltpu.sync_copy(im_v, im_out.at[pl.ds(base, CM)])


def _sc_gmf_body(uk_hbm, ik_hbm, pu_tab, pi_tab, dep_hbm, pu_out, pi_out,
                 idx_u, idx_i, pu_v, pi_v, s1, s2):
    wid = lax.axis_index("s") * NC + lax.axis_index("c")
    for c in range(BW // CM):
        base = wid * BW + c * CM
        pltpu.sync_copy(uk_hbm.at[pl.ds(base, CM)], idx_u)
        pltpu.sync_copy(ik_hbm.at[pl.ds(base, CM)], idx_i)
        cp_pu = pltpu.async_copy(pu_tab.at[idx_u], pu_v, s1)
        cp_pi = pltpu.async_copy(pi_tab.at[idx_i], pi_v, s2)
        cp_pu.wait()
        pltpu.sync_copy(pu_v, pu_out.at[pl.ds(base, CM)])
        cp_pi.wait()
        pltpu.sync_copy(pi_v, pi_out.at[pl.ds(base, CM)])


def _mesh():
    return plsc.VectorSubcoreMesh(core_axis_name="c", subcore_axis_name="s",
                                  num_cores=NC, num_subcores=NS)


def _sc_gather_mlp(user, item, um_tab, im_tab):
    f32 = jnp.float32
    fn = pl.kernel(
        _sc_mlp_body,
        out_type=[
            jax.ShapeDtypeStruct((B, DM), f32),
            jax.ShapeDtypeStruct((B, DM), f32),
        ],
        mesh=_mesh(),
        scratch_types=[
            pltpu.VMEM((CM,), jnp.int32),
            pltpu.VMEM((CM,), jnp.int32),
            pltpu.VMEM((CM, DM), f32),
            pltpu.VMEM((CM, DM), f32),
            pltpu.SemaphoreType.DMA,
            pltpu.SemaphoreType.DMA,
        ],
    )
    return fn(user, item, um_tab, im_tab)


def _sc_gather_gmf(uk, ik, pu_tab, pi_tab, dep):
    f32 = jnp.float32
    fn = pl.kernel(
        _sc_gmf_body,
        out_type=[
            jax.ShapeDtypeStruct((B, DP), f32),
            jax.ShapeDtypeStruct((B, DP), f32),
        ],
        mesh=_mesh(),
        scratch_types=[
            pltpu.VMEM((CM,), jnp.int32),
            pltpu.VMEM((CM,), jnp.int32),
            pltpu.VMEM((CM, DP), f32),
            pltpu.VMEM((CM, DP), f32),
            pltpu.SemaphoreType.DMA,
            pltpu.SemaphoreType.DMA,
        ],
    )
    return fn(uk, ik, pu_tab, pi_tab, dep)


TB = 1024


def _unpack_half(words, hb):
    u = jnp.uint32
    w = jax.lax.bitcast_convert_type(words, u)
    lo = jax.lax.bitwise_and(w, jnp.uint32(0xFFFF))
    hi = jax.lax.shift_right_logical(w, jnp.uint32(16))
    sel = jnp.where(hb > 0.5, hi, lo).astype(jnp.uint16)
    return jax.lax.bitcast_convert_type(sel, jnp.bfloat16).astype(jnp.float32)


def _tc_mlp_body(pur, pir, uha, uhb, iha, ihb, um, im, w1a, w1b, b1, w2,
                 b2, w3, b3, wpb, bp, out):
    f32 = jnp.float32
    hp = jax.lax.Precision.DEFAULT
    h = jnp.dot(um[...], w1a[...], precision=hp, preferred_element_type=f32)
    h += jnp.dot(im[...], w1b[...], precision=hp, preferred_element_type=f32)
    h = jnp.maximum(h + b1[...], 0.0)
    h = jnp.maximum(
        jnp.dot(h, w2[...], precision=hp, preferred_element_type=f32)
        + b2[...], 0.0)
    h = jnp.maximum(
        jnp.dot(h, w3[...], precision=hp, preferred_element_type=f32)
        + b3[...], 0.0)
    pu = pur[...]
    pi = pir[...]
    pu = jnp.where(uha[...] > 0.5, pu[:, DG:], pu[:, :DG])
    pi = jnp.where(iha[...] > 0.5, pi[:, DG:], pi[:, :DG])
    ug = _unpack_half(pu, uhb[...])
    ig = _unpack_half(pi, ihb[...])
    pred = (jnp.sum(ug * ig, axis=1) + jnp.sum(h * wpb[...], axis=1)
            + bp[0, 0])
    out[...] = pred


def _tc_mlp(pur, pir, uha, uhb, iha, ihb, um, im, W1, b1, W2, b2, W3, b3,
            Wp, bp):
    f32 = jnp.float32
    w1a, w1b = W1[:DM], W1[DM:]
    wpb = Wp[DG:, 0].reshape(1, DG)
    grid = (B // TB,)
    full = lambda i: (0, 0)
    return pl.pallas_call(
        _tc_mlp_body,
        grid=grid,
        in_specs=[
            pl.BlockSpec((TB, DP), lambda i: (i, 0)),
            pl.BlockSpec((TB, DP), lambda i: (i, 0)),
            pl.BlockSpec((TB, 1), lambda i: (i, 0)),
            pl.BlockSpec((TB, 1), lambda i: (i, 0)),
            pl.BlockSpec((TB, 1), lambda i: (i, 0)),
            pl.BlockSpec((TB, 1), lambda i: (i, 0)),
            pl.BlockSpec((TB, DM), lambda i: (i, 0)),
            pl.BlockSpec((TB, DM), lambda i: (i, 0)),
            pl.BlockSpec((DM, 256), full),
            pl.BlockSpec((DM, 256), full),
            pl.BlockSpec((1, 256), full),
            pl.BlockSpec((256, 128), full),
            pl.BlockSpec((1, 128), full),
            pl.BlockSpec((128, DG), full),
            pl.BlockSpec((1, DG), full),
            pl.BlockSpec((1, DG), full),
            pl.BlockSpec((1, 1), full),
        ],
        out_specs=pl.BlockSpec((TB,), lambda i: (i,)),
        out_shape=jax.ShapeDtypeStruct((B,), f32),
    )(pur, pir, uha, uhb, iha, ihb, um, im, w1a, w1b, b1.reshape(1, 256),
      W2, b2.reshape(1, 128), W3, b3.reshape(1, DG), wpb, bp.reshape(1, 1))


def kernel(user, item, embed_user_GMF, embed_item_GMF, embed_user_MLP,
           embed_item_MLP, W1, b1, W2, b2, W3, b3, Wp, bp):
    f32 = jnp.float32
    um, im = _sc_gather_mlp(user, item, embed_user_MLP, embed_item_MLP)

    ugT = embed_user_GMF.T
    igT = embed_item_GMF.T
    pu_tab, pi_tab = _tc_pack(ugT, igT, Wp[:DG, 0].reshape(1, DG))

    sh = CB.bit_length() - 1

    def flags(idx):
        blk = jax.lax.shift_right_logical(idx, sh)
        k = blk * (CB // 4) + jax.lax.bitwise_and(idx, (CB // 4) - 1)
        ha = jax.lax.convert_element_type(
            jax.lax.bitwise_and(jax.lax.shift_right_logical(idx, sh - 2), 1),
            f32).reshape(B, 1)
        hb = jax.lax.convert_element_type(
            jax.lax.bitwise_and(jax.lax.shift_right_logical(idx, sh - 1), 1),
            f32).reshape(B, 1)
        return k, ha, hb

    uk, uha, uhb = flags(user)
    ik, iha, ihb = flags(item)

    pur, pir = _sc_gather_gmf(uk, ik, pu_tab, pi_tab, um)
    return _tc_mlp(pur, pir, uha, uhb, iha, ihb, um, im, W1, b1, W2, b2,
                   W3, b3, Wp, bp)
